# baseline (speedup 1.0000x reference)
import functools

import jax
import jax.numpy as jnp
from jax import lax
from jax.experimental import pallas as pl
from jax.experimental.pallas import tpu as pltpu
from jax.experimental.pallas import tpu_sc as plsc

N_NODES = 10000
N_EDGES = 320000
D = 128

NC = 2
NS = 16
NW = NC * NS

NPAD = 10240
RPT = NPAD // NS
C = 128
EPT = 10240
E_PAD = NW * EPT
ITERS = EPT // C
DEGW = 128


def _sc_sums(xpad, src, dst):
    mesh = plsc.VectorSubcoreMesh(core_axis_name="c", subcore_axis_name="s",
                                  num_cores=NC, num_subcores=NS)

    @functools.partial(
        pl.kernel,
        out_type=jax.ShapeDtypeStruct((NC * NPAD, D), jnp.float32),
        mesh=mesh,
        name="sc_sums",
        scratch_types=[
            pltpu.VMEM((C,), jnp.int32),
            pltpu.VMEM((C,), jnp.int32),
            pltpu.VMEM((C, D), jnp.float32),
            pltpu.VMEM((C,), jnp.int32),
            pltpu.VMEM((C,), jnp.int32),
            pltpu.VMEM((C, D), jnp.float32),
            pltpu.SemaphoreType.DMA,
            pltpu.SemaphoreType.DMA,
            pltpu.VMEM_SHARED((NPAD, D), jnp.float32),
        ],
    )
    def sums_kernel(x_hbm, src_hbm, dst_hbm, sum_out,
                    sa, da, ra, sb, db, rb, sema, semb, acc):
        c = lax.axis_index("c")
        s = lax.axis_index("s")
        w = c * NS + s
        ebase = w * EPT

        @pl.when(c == 0)
        def _initx():
            pltpu.sync_copy(x_hbm.at[pl.ds(s * RPT, RPT)],
                            acc.at[pl.ds(s * RPT, RPT)])

        @pl.when(c == 1)
        def _initz():
            @pl.loop(0, RPT // 80)
            def _z(k):
                pltpu.sync_copy(x_hbm.at[pl.ds(N_NODES, 80)],
                                acc.at[pl.ds(s * RPT + k * 80, 80)])

        plsc.subcore_barrier()

        def load_idx(it, sref, dref):
            pltpu.sync_copy(src_hbm.at[pl.ds(ebase + it * C, C)], sref)
            pltpu.sync_copy(dst_hbm.at[pl.ds(ebase + it * C, C)], dref)

        load_idx(0, sa, da)
        pltpu.async_copy(x_hbm.at[sa], ra, sema)

        @pl.loop(0, ITERS // 2 - 1)
        def _edges(p):
            load_idx(2 * p + 1, sb, db)
            gb = pltpu.async_copy(x_hbm.at[sb], rb, semb)
            pltpu.make_async_copy(x_hbm.at[pl.ds(0, C)], ra, sema).wait()
            pltpu.sync_copy(ra, acc.at[da], add=True)
            load_idx(2 * p + 2, sa, da)
            pltpu.async_copy(x_hbm.at[sa], ra, sema)
            gb.wait()
            pltpu.sync_copy(rb, acc.at[db], add=True)

        load_idx(ITERS - 1, sb, db)
        gb = pltpu.async_copy(x_hbm.at[sb], rb, semb)
        pltpu.make_async_copy(x_hbm.at[pl.ds(0, C)], ra, sema).wait()
        pltpu.sync_copy(ra, acc.at[da], add=True)
        gb.wait()
        pltpu.sync_copy(rb, acc.at[db], add=True)

        plsc.subcore_barrier()

        pltpu.sync_copy(acc.at[pl.ds(s * RPT, RPT)],
                        sum_out.at[pl.ds(c * NPAD + s * RPT, RPT)])

    return sums_kernel(xpad, src, dst)


def _sc_degree(xpad, dst):
    mesh = plsc.VectorSubcoreMesh(core_axis_name="c", subcore_axis_name="s",
                                  num_cores=NC, num_subcores=NS)

    @functools.partial(
        pl.kernel,
        out_type=jax.ShapeDtypeStruct((NC * NPAD, DEGW), jnp.float32),
        mesh=mesh,
        name="sc_deg",
        scratch_types=[
            pltpu.VMEM((C, DEGW), jnp.float32),
            pltpu.VMEM((C,), jnp.int32),
            pltpu.VMEM((C,), jnp.int32),
            pltpu.SemaphoreType.DMA,
            pltpu.SemaphoreType.DMA,
            pltpu.VMEM_SHARED((NPAD, DEGW), jnp.float32),
        ],
    )
    def deg_kernel(x_hbm, dst_hbm, deg_out, ones, da, db, sema, semb, dacc):
        c = lax.axis_index("c")
        s = lax.axis_index("s")
        w = c * NS + s
        ebase = w * EPT

        zero16 = jnp.zeros((16,), jnp.float32)
        onehot = jnp.where(lax.iota(jnp.int32, 16) == 0,
                           jnp.float32(1), jnp.float32(0))

        @pl.loop(0, C)
        def _fill(i):
            ones[i, pl.ds(0, 16)] = onehot
            for j in range(1, DEGW // 16):
                ones[i, pl.ds(j * 16, 16)] = zero16

        @pl.loop(0, RPT // 80)
        def _zero(k):
            pltpu.sync_copy(x_hbm.at[pl.ds(N_NODES, 80)],
                            dacc.at[pl.ds(s * RPT + k * 80, 80)])

        plsc.subcore_barrier()

        pltpu.async_copy(dst_hbm.at[pl.ds(ebase, C)], da, sema)

        @pl.loop(0, ITERS // 2 - 1)
        def _edges(p):
            pltpu.async_copy(
                dst_hbm.at[pl.ds(ebase + (2 * p + 1) * C, C)], db, semb)
            pltpu.make_async_copy(dst_hbm.at[pl.ds(0, C)], da, sema).wait()
            pltpu.sync_copy(ones, dacc.at[da], add=True)
            pltpu.async_copy(
                dst_hbm.at[pl.ds(ebase + (2 * p + 2) * C, C)], da, sema)
            pltpu.make_async_copy(dst_hbm.at[pl.ds(0, C)], db, semb).wait()
            pltpu.sync_copy(ones, dacc.at[db], add=True)

        pltpu.async_copy(
            dst_hbm.at[pl.ds(ebase + (ITERS - 1) * C, C)], db, semb)
        pltpu.make_async_copy(dst_hbm.at[pl.ds(0, C)], da, sema).wait()
        pltpu.sync_copy(ones, dacc.at[da], add=True)
        pltpu.make_async_copy(dst_hbm.at[pl.ds(0, C)], db, semb).wait()
        pltpu.sync_copy(ones, dacc.at[db], add=True)

        plsc.subcore_barrier()

        pltpu.sync_copy(dacc.at[pl.ds(s * RPT, RPT)],
                        deg_out.at[pl.ds(c * NPAD + s * RPT, RPT)])

    return deg_kernel(xpad, dst)


BR = 256


def _tc_combine_body(s0_ref, s1_ref, d0_ref, d1_ref, w_ref, b_ref, o_ref):
    total = s0_ref[...] + s1_ref[...]
    deg = d0_ref[:, 0:1] + d1_ref[:, 0:1] + 1.0
    agg = total / deg
    h = jnp.dot(agg, w_ref[...], preferred_element_type=jnp.float32)
    h = jnp.maximum(h + b_ref[...], 0.0)
    n = jnp.sqrt(jnp.sum(h * h, axis=1, keepdims=True))
    o_ref[...] = h / jnp.maximum(n, 1e-12)


def _tc_combine(sums, degs, W, b2):
    grid = NPAD // BR
    return pl.pallas_call(
        _tc_combine_body,
        grid=(grid,),
        in_specs=[
            pl.BlockSpec((BR, D), lambda i: (i, 0)),
            pl.BlockSpec((BR, D), lambda i, _g=grid: (i + _g, 0)),
            pl.BlockSpec((BR, DEGW), lambda i: (i, 0)),
            pl.BlockSpec((BR, DEGW), lambda i, _g=grid: (i + _g, 0)),
            pl.BlockSpec((D, D), lambda i: (0, 0)),
            pl.BlockSpec((1, D), lambda i: (0, 0)),
        ],
        out_specs=pl.BlockSpec((BR, D), lambda i: (i, 0)),
        out_shape=jax.ShapeDtypeStruct((NPAD, D), jnp.float32),
    )(sums, sums, degs, degs, W, b2)


def kernel(x, edge_index, W, b):
    pad_e = E_PAD - N_EDGES
    src = jnp.concatenate([edge_index[0],
                           jnp.zeros((pad_e,), jnp.int32)])
    dst = jnp.concatenate([edge_index[1],
                           jnp.full((pad_e,), NPAD - 1, jnp.int32)])
    xpad = jnp.pad(x, ((0, NPAD - N_NODES), (0, 0)))
    sums = _sc_sums(xpad, src, dst)
    degs = _sc_degree(xpad, dst)
    h = _tc_combine(sums, degs, W, b.reshape(1, D))
    return h[:N_NODES]


# speedup vs baseline: 1.0281x; 1.0281x over previous
import functools

import jax
import jax.numpy as jnp
from jax import lax
from jax.experimental import pallas as pl
from jax.experimental.pallas import tpu as pltpu
from jax.experimental.pallas import tpu_sc as plsc

N_NODES = 10000
N_EDGES = 320000
D = 128

NC = 2
NS = 16
NW = NC * NS

NPAD = 10240
RPT = NPAD // NS
C = 128
EPT = 10240
E_PAD = NW * EPT
ITERS = EPT // C
DEGW = 128


def _sc_sums(xpad, zpad, src, dst):
    mesh = plsc.VectorSubcoreMesh(core_axis_name="c", subcore_axis_name="s",
                                  num_cores=NC, num_subcores=NS)

    @functools.partial(
        pl.kernel,
        out_type=jax.ShapeDtypeStruct((NC * NPAD, D), jnp.float32),
        mesh=mesh,
        name="sc_sums",
        scratch_types=[
            pltpu.VMEM((C,), jnp.int32),
            pltpu.VMEM((C,), jnp.int32),
            pltpu.VMEM((C, D), jnp.float32),
            pltpu.VMEM((C,), jnp.int32),
            pltpu.VMEM((C,), jnp.int32),
            pltpu.VMEM((C, D), jnp.float32),
            pltpu.SemaphoreType.DMA,
            pltpu.SemaphoreType.DMA,
            pltpu.VMEM_SHARED((NPAD, D), jnp.float32),
        ],
    )
    def sums_kernel(x_hbm, z_hbm, src_hbm, dst_hbm, sum_out,
                    sa, da, ra, sb, db, rb, sema, semb, acc):
        c = lax.axis_index("c")
        s = lax.axis_index("s")
        w = c * NS + s
        ebase = w * EPT

        @pl.when(c == 0)
        def _initx():
            pltpu.sync_copy(x_hbm.at[pl.ds(s * RPT, RPT)],
                            acc.at[pl.ds(s * RPT, RPT)])

        @pl.when(c == 1)
        def _initz():
            pltpu.sync_copy(z_hbm.at[pl.ds(s * RPT, RPT)],
                            acc.at[pl.ds(s * RPT, RPT)])

        plsc.subcore_barrier()

        def load_idx(it, sref, dref):
            pltpu.sync_copy(src_hbm.at[pl.ds(ebase + it * C, C)], sref)
            pltpu.sync_copy(dst_hbm.at[pl.ds(ebase + it * C, C)], dref)

        load_idx(0, sa, da)
        pltpu.async_copy(x_hbm.at[sa], ra, sema)

        @pl.loop(0, ITERS // 2 - 1)
        def _edges(p):
            load_idx(2 * p + 1, sb, db)
            gb = pltpu.async_copy(x_hbm.at[sb], rb, semb)
            pltpu.make_async_copy(x_hbm.at[pl.ds(0, C)], ra, sema).wait()
            pltpu.sync_copy(ra, acc.at[da], add=True)
            load_idx(2 * p + 2, sa, da)
            pltpu.async_copy(x_hbm.at[sa], ra, sema)
            gb.wait()
            pltpu.sync_copy(rb, acc.at[db], add=True)

        load_idx(ITERS - 1, sb, db)
        gb = pltpu.async_copy(x_hbm.at[sb], rb, semb)
        pltpu.make_async_copy(x_hbm.at[pl.ds(0, C)], ra, sema).wait()
        pltpu.sync_copy(ra, acc.at[da], add=True)
        gb.wait()
        pltpu.sync_copy(rb, acc.at[db], add=True)

        plsc.subcore_barrier()

        pltpu.sync_copy(acc.at[pl.ds(s * RPT, RPT)],
                        sum_out.at[pl.ds(c * NPAD + s * RPT, RPT)])

    return sums_kernel(xpad, zpad, src, dst)


def _sc_degree(zpad, dst):
    mesh = plsc.VectorSubcoreMesh(core_axis_name="c", subcore_axis_name="s",
                                  num_cores=NC, num_subcores=NS)

    @functools.partial(
        pl.kernel,
        out_type=jax.ShapeDtypeStruct((NC * NPAD, DEGW), jnp.float32),
        mesh=mesh,
        name="sc_deg",
        scratch_types=[
            pltpu.VMEM((C, DEGW), jnp.float32),
            pltpu.VMEM((C,), jnp.int32),
            pltpu.VMEM((C,), jnp.int32),
            pltpu.SemaphoreType.DMA,
            pltpu.SemaphoreType.DMA,
            pltpu.VMEM_SHARED((NPAD, DEGW), jnp.float32),
        ],
    )
    def deg_kernel(x_hbm, dst_hbm, deg_out, ones, da, db, sema, semb, dacc):
        c = lax.axis_index("c")
        s = lax.axis_index("s")
        w = c * NS + s
        ebase = w * EPT

        zero16 = jnp.zeros((16,), jnp.float32)
        onehot = jnp.where(lax.iota(jnp.int32, 16) == 0,
                           jnp.float32(1), jnp.float32(0))

        @pl.loop(0, C)
        def _fill(i):
            ones[i, pl.ds(0, 16)] = onehot
            for j in range(1, DEGW // 16):
                ones[i, pl.ds(j * 16, 16)] = zero16

        pltpu.sync_copy(x_hbm.at[pl.ds(s * RPT, RPT)],
                        dacc.at[pl.ds(s * RPT, RPT)])

        plsc.subcore_barrier()

        pltpu.async_copy(dst_hbm.at[pl.ds(ebase, C)], da, sema)

        @pl.loop(0, ITERS // 2 - 1)
        def _edges(p):
            pltpu.async_copy(
                dst_hbm.at[pl.ds(ebase + (2 * p + 1) * C, C)], db, semb)
            pltpu.make_async_copy(dst_hbm.at[pl.ds(0, C)], da, sema).wait()
            pltpu.sync_copy(ones, dacc.at[da], add=True)
            pltpu.async_copy(
                dst_hbm.at[pl.ds(ebase + (2 * p + 2) * C, C)], da, sema)
            pltpu.make_async_copy(dst_hbm.at[pl.ds(0, C)], db, semb).wait()
            pltpu.sync_copy(ones, dacc.at[db], add=True)

        pltpu.async_copy(
            dst_hbm.at[pl.ds(ebase + (ITERS - 1) * C, C)], db, semb)
        pltpu.make_async_copy(dst_hbm.at[pl.ds(0, C)], da, sema).wait()
        pltpu.sync_copy(ones, dacc.at[da], add=True)
        pltpu.make_async_copy(dst_hbm.at[pl.ds(0, C)], db, semb).wait()
        pltpu.sync_copy(ones, dacc.at[db], add=True)

        plsc.subcore_barrier()

        pltpu.sync_copy(dacc.at[pl.ds(s * RPT, RPT)],
                        deg_out.at[pl.ds(c * NPAD + s * RPT, RPT)])

    return deg_kernel(zpad, dst)


BR = 256


def _tc_combine_body(s0_ref, s1_ref, d0_ref, d1_ref, w_ref, b_ref, o_ref):
    total = s0_ref[...] + s1_ref[...]
    deg = d0_ref[:, 0:1] + d1_ref[:, 0:1] + 1.0
    agg = total / deg
    h = jnp.dot(agg, w_ref[...], preferred_element_type=jnp.float32)
    h = jnp.maximum(h + b_ref[...], 0.0)
    n = jnp.sqrt(jnp.sum(h * h, axis=1, keepdims=True))
    o_ref[...] = h / jnp.maximum(n, 1e-12)


def _tc_combine(sums, degs, W, b2):
    grid = NPAD // BR
    return pl.pallas_call(
        _tc_combine_body,
        grid=(grid,),
        in_specs=[
            pl.BlockSpec((BR, D), lambda i: (i, 0)),
            pl.BlockSpec((BR, D), lambda i, _g=grid: (i + _g, 0)),
            pl.BlockSpec((BR, DEGW), lambda i: (i, 0)),
            pl.BlockSpec((BR, DEGW), lambda i, _g=grid: (i + _g, 0)),
            pl.BlockSpec((D, D), lambda i: (0, 0)),
            pl.BlockSpec((1, D), lambda i: (0, 0)),
        ],
        out_specs=pl.BlockSpec((BR, D), lambda i: (i, 0)),
        out_shape=jax.ShapeDtypeStruct((NPAD, D), jnp.float32),
    )(sums, sums, degs, degs, W, b2)


def kernel(x, edge_index, W, b):
    pad_e = E_PAD - N_EDGES
    src = jnp.concatenate([edge_index[0],
                           jnp.zeros((pad_e,), jnp.int32)])
    dst = jnp.concatenate([edge_index[1],
                           jnp.full((pad_e,), NPAD - 1, jnp.int32)])
    xpad = jnp.pad(x, ((0, NPAD - N_NODES), (0, 0)))
    zpad = jnp.zeros((NPAD, D), jnp.float32)
    sums = _sc_sums(xpad, zpad, src, dst)
    degs = _sc_degree(zpad, dst)
    h = _tc_combine(sums, degs, W, b.reshape(1, D))
    return h[:N_NODES]
